# Initial kernel scaffold; baseline (speedup 1.0000x reference)
import jax
import jax.numpy as jnp
from jax.experimental import pallas as pl


def kernel(x, b_pre, W_enc, b_enc, W_dec):
    raise NotImplementedError("write your pallas kernel here")


# speedup vs baseline: 7.1012x; 7.1012x over previous
import functools

import jax
import jax.numpy as jnp
from jax.experimental import pallas as pl
from jax.experimental.pallas import tpu as pltpu

_K = 64


def _encode_select_body(x_ref, bpre_ref, W_ref, benc_ref, out_ref, *, nj, hb):
    j = pl.program_id(1)

    @pl.when(j < nj)
    def _compute_block():
        xc = x_ref[...] - bpre_ref[...]
        h = jax.lax.dot_general(
            xc, W_ref[...], (((1,), (1,)), ((), ())),
            preferred_element_type=jnp.float32)
        base = pl.multiple_of(j * hb, hb)
        out_ref[:, pl.ds(base, hb)] = h + benc_ref[...]

    @pl.when(j == nj)
    def _select():
        h = out_ref[...]
        key = jax.lax.bitcast_convert_type(h, jnp.int32)
        key = jnp.where(key < 0, key ^ jnp.int32(0x7FFFFFFF), key)
        rows = h.shape[0]
        lo0 = jnp.full((rows, 1), jnp.int32(-2147483647 - 1))
        hi0 = jnp.full((rows, 1), jnp.int32(2147483647))

        def body(_, carry):
            lo, hi = carry
            mid = (lo >> 1) + (hi >> 1) + ((lo & 1) | (hi & 1))
            cnt = jnp.sum((key >= mid).astype(jnp.int32), axis=1,
                          keepdims=True)
            take = cnt >= _K
            return jnp.where(take, mid, lo), jnp.where(take, hi, mid - 1)

        lo, _ = jax.lax.fori_loop(0, 32, body, (lo0, hi0))
        mask = key >= lo
        out_ref[...] = jnp.where(mask, jnp.maximum(h, 0.0), 0.0)


def _decode_body(hs_ref, W_ref, bpre_ref, out_ref):
    k = pl.program_id(1)

    @pl.when(k == 0)
    def _init():
        out_ref[...] = jnp.broadcast_to(bpre_ref[...], out_ref.shape)

    out_ref[...] += jax.lax.dot_general(
        hs_ref[...], W_ref[...], (((1,), (1,)), ((), ())),
        preferred_element_type=jnp.float32)


def kernel(x, b_pre, W_enc, b_enc, W_dec):
    bsz, d = x.shape
    hdim = W_enc.shape[0]

    r1 = min(128, bsz)
    hb = min(1024, hdim)
    nj = hdim // hb

    h_sparse = pl.pallas_call(
        functools.partial(_encode_select_body, nj=nj, hb=hb),
        grid=(bsz // r1, nj + 1),
        in_specs=[
            pl.BlockSpec((r1, d), lambda i, j: (i, 0)),
            pl.BlockSpec((1, d), lambda i, j: (0, 0)),
            pl.BlockSpec((hb, d), lambda i, j: (jnp.minimum(j, nj - 1), 0)),
            pl.BlockSpec((1, hb), lambda i, j: (0, jnp.minimum(j, nj - 1))),
        ],
        out_specs=pl.BlockSpec((r1, hdim), lambda i, j: (i, 0)),
        out_shape=jax.ShapeDtypeStruct((bsz, hdim), jnp.float32),
    )(x, b_pre[None], W_enc, b_enc[None])

    r2 = min(1024, bsz)
    kb = min(1024, hdim)
    nk = hdim // kb
    x_rec = pl.pallas_call(
        _decode_body,
        grid=(bsz // r2, nk),
        in_specs=[
            pl.BlockSpec((r2, kb), lambda i, k: (i, k)),
            pl.BlockSpec((d, kb), lambda i, k: (0, k)),
            pl.BlockSpec((1, d), lambda i, k: (0, 0)),
        ],
        out_specs=pl.BlockSpec((r2, d), lambda i, k: (i, 0)),
        out_shape=jax.ShapeDtypeStruct((bsz, d), jnp.float32),
    )(h_sparse, W_dec, b_pre[None])

    return (x_rec, h_sparse)


# speedup vs baseline: 8.3323x; 1.1734x over previous
import functools

import jax
import jax.numpy as jnp
from jax.experimental import pallas as pl
from jax.experimental.pallas import tpu as pltpu

_K = 64


def _encode_select_body(x_ref, bpre_ref, W_ref, benc_ref, out_ref,
                        scr_ref, lo_ref, hi_ref, *, ni, nj, hb):
    i = pl.program_id(0)
    j = pl.program_id(1)

    @pl.when(i < ni)
    def _matmul_block():
        xc = x_ref[...] - bpre_ref[...]
        h = jax.lax.dot_general(
            xc, W_ref[...], (((1,), (1,)), ((), ())),
            preferred_element_type=jnp.float32)
        h = h + benc_ref[...]
        iu = jax.lax.bitcast_convert_type(h, jnp.int32)
        key = jnp.where(iu < 0, iu ^ jnp.int32(0x7FFFFFFF), iu)
        base = pl.multiple_of(j * hb, hb)
        scr_ref[i % 2, :, pl.ds(base, hb)] = key

    @pl.when(i > 0)
    def _select_prev():
        @pl.when(j == 0)
        def _init():
            lo_ref[...] = jnp.full(lo_ref.shape, jnp.int32(-2147483647 - 1))
            hi_ref[...] = jnp.full(hi_ref.shape, jnp.int32(2147483647))

        key = scr_ref[(i - 1) % 2]
        for _ in range(-(-32 // nj)):
            lo = lo_ref[...]
            hi = hi_ref[...]
            mid = (lo >> 1) + (hi >> 1) + ((lo & 1) | (hi & 1))
            cnt = jnp.sum((key >= mid).astype(jnp.int32), axis=1,
                          keepdims=True)
            take = cnt >= _K
            lo_ref[...] = jnp.where(take, mid, lo)
            hi_ref[...] = jnp.where(take, hi, mid - 1)

        @pl.when(j == nj - 1)
        def _write():
            thr = lo_ref[...]
            k2 = scr_ref[(i - 1) % 2]
            mask = k2 >= thr
            h = jax.lax.bitcast_convert_type(
                jnp.where(k2 < 0, k2 ^ jnp.int32(0x7FFFFFFF), k2),
                jnp.float32)
            out_ref[...] = jnp.where(mask, jnp.maximum(h, 0.0), 0.0)


def _decode_body(hs_ref, W_ref, bpre_ref, out_ref):
    k = pl.program_id(1)

    @pl.when(k == 0)
    def _init():
        out_ref[...] = jnp.broadcast_to(bpre_ref[...], out_ref.shape)

    out_ref[...] += jax.lax.dot_general(
        hs_ref[...], W_ref[...], (((1,), (1,)), ((), ())),
        preferred_element_type=jnp.float32)


def kernel(x, b_pre, W_enc, b_enc, W_dec):
    bsz, d = x.shape
    hdim = W_enc.shape[0]

    r1 = min(128, bsz)
    hb = min(1024, hdim)
    nj = hdim // hb
    ni = bsz // r1

    h_sparse = pl.pallas_call(
        functools.partial(_encode_select_body, ni=ni, nj=nj, hb=hb),
        grid=(ni + 1, nj),
        in_specs=[
            pl.BlockSpec((r1, d), lambda i, j: (jnp.minimum(i, ni - 1), 0)),
            pl.BlockSpec((1, d), lambda i, j: (0, 0)),
            pl.BlockSpec((hb, d), lambda i, j: (j, 0)),
            pl.BlockSpec((1, hb), lambda i, j: (0, j)),
        ],
        out_specs=pl.BlockSpec((r1, hdim),
                               lambda i, j: (jnp.maximum(i - 1, 0), 0)),
        out_shape=jax.ShapeDtypeStruct((bsz, hdim), jnp.float32),
        scratch_shapes=[
            pltpu.VMEM((2, r1, hdim), jnp.int32),
            pltpu.VMEM((r1, 1), jnp.int32),
            pltpu.VMEM((r1, 1), jnp.int32),
        ],
    )(x, b_pre[None], W_enc, b_enc[None])

    r2 = min(1024, bsz)
    kb = min(1024, hdim)
    nk = hdim // kb
    x_rec = pl.pallas_call(
        _decode_body,
        grid=(bsz // r2, nk),
        in_specs=[
            pl.BlockSpec((r2, kb), lambda i, k: (i, k)),
            pl.BlockSpec((d, kb), lambda i, k: (0, k)),
            pl.BlockSpec((1, d), lambda i, k: (0, 0)),
        ],
        out_specs=pl.BlockSpec((r2, d), lambda i, k: (i, 0)),
        out_shape=jax.ShapeDtypeStruct((bsz, d), jnp.float32),
    )(h_sparse, W_dec, b_pre[None])

    return (x_rec, h_sparse)
